# Initial kernel scaffold; baseline (speedup 1.0000x reference)
import jax
import jax.numpy as jnp
from jax.experimental import pallas as pl


def kernel(features, positions, gene_ids, mask, W, b):
    raise NotImplementedError("write your pallas kernel here")


# speedup vs baseline: 546.8864x; 546.8864x over previous
import functools

import jax
import jax.numpy as jnp
from jax import lax
from jax.experimental import pallas as pl
from jax.experimental.pallas import tpu as pltpu
from jax.experimental.pallas import tpu_sc as plsc

_NUM_GENES = 20000
_B = 64
_V = 512
_D = 64


def _make_sc_gather(num_rows, d, n_idx):
    info = plsc.get_sparse_core_info()
    nw = info.num_cores * info.num_subcores
    per_w = n_idx // nw
    mesh = plsc.VectorSubcoreMesh(core_axis_name="c", subcore_axis_name="s")

    @functools.partial(
        pl.kernel,
        mesh=mesh,
        out_type=jax.ShapeDtypeStruct((n_idx, d), jnp.float32),
        scratch_types=[
            pltpu.VMEM((per_w,), jnp.int32),
            pltpu.VMEM((per_w, d), jnp.float32),
            pltpu.SemaphoreType.DMA,
        ],
        compiler_params=pltpu.CompilerParams(use_tc_tiling_on_sc=False),
    )
    def sc_gather(table_hbm, idx_hbm, out_hbm, idx_v, rows_v, sem):
        wid = lax.axis_index("s") * info.num_cores + lax.axis_index("c")
        base = wid * per_w
        pltpu.sync_copy(idx_hbm.at[pl.ds(base, per_w)], idx_v)
        pltpu.async_copy(table_hbm.at[idx_v], rows_v, sem).wait()
        pltpu.sync_copy(rows_v, out_hbm.at[pl.ds(base, per_w)])

    return sc_gather


def kernel(features, positions, gene_ids, mask, W, b):
    del positions
    B, V, D = features.shape

    table = W.reshape(_NUM_GENES, _D)
    flat_idx = jnp.minimum(gene_ids, _NUM_GENES - 1).reshape(B * V)

    gathered = _make_sc_gather(_NUM_GENES, D, B * V)(table, flat_idx)
    gathered = gathered.reshape(B, V, D)

    gid_row = gene_ids.reshape(B, 1, V)
    gid_col = gene_ids.reshape(B, V, 1)
    msk_row = mask.reshape(B, 1, V)
    msk_col = mask.reshape(B, V, 1)
    b2 = b.reshape(1, 1)

    def tc_body(feat_ref, gath_ref, gr_ref, gc_ref, mr_ref, mc_ref,
                b_ref, out_ref):
        f = feat_ref[0]
        g = gath_ref[0]
        grow = gr_ref[0]
        gcol = gc_ref[0]
        mrow = mr_ref[0]
        mcol = mc_ref[0]
        valid_col = (mcol > 0) & (gcol < _NUM_GENES)
        valid_row = (mrow > 0) & (grow < _NUM_GENES)
        eq = gcol == grow
        j_idx = lax.broadcasted_iota(jnp.int32, (V, V), 1)
        v_idx = lax.broadcasted_iota(jnp.int32, (V, V), 0)
        later = j_idx > v_idx
        killed = jnp.any(eq & later & valid_row, axis=1, keepdims=True)
        winner = valid_col & jnp.logical_not(killed)
        dots = jnp.sum(f * g, axis=1, keepdims=True)
        contrib = jnp.sum(jnp.where(winner, dots, 0.0))
        out_ref[0] = jnp.full((1, 128), contrib + b_ref[0, 0], jnp.float32)

    V_, D_ = V, D
    out = pl.pallas_call(
        tc_body,
        grid=(B,),
        in_specs=[
            pl.BlockSpec((1, V_, D_), lambda i: (i, 0, 0)),
            pl.BlockSpec((1, V_, D_), lambda i: (i, 0, 0)),
            pl.BlockSpec((1, 1, V_), lambda i: (i, 0, 0)),
            pl.BlockSpec((1, V_, 1), lambda i: (i, 0, 0)),
            pl.BlockSpec((1, 1, V_), lambda i: (i, 0, 0)),
            pl.BlockSpec((1, V_, 1), lambda i: (i, 0, 0)),
            pl.BlockSpec((1, 1), lambda i: (0, 0)),
        ],
        out_specs=pl.BlockSpec((1, 1, 128), lambda i: (i, 0, 0)),
        out_shape=jax.ShapeDtypeStruct((B, 1, 128), jnp.float32),
    )(features, gathered, gid_row, gid_col, msk_row, msk_col, b2)
    return out[:, 0, :1]


# speedup vs baseline: 654.0461x; 1.1959x over previous
import functools

import jax
import jax.numpy as jnp
from jax import lax
from jax.experimental import pallas as pl
from jax.experimental.pallas import tpu as pltpu
from jax.experimental.pallas import tpu_sc as plsc

_NUM_GENES = 20000
_B = 64
_V = 512
_D = 64
_PAIRS = _B * _V


def _make_sc_dots():
    info = plsc.get_sparse_core_info()
    nw = info.num_cores * info.num_subcores
    per_w = _PAIRS // nw
    chunk = per_w // 4
    mesh = plsc.VectorSubcoreMesh(core_axis_name="c", subcore_axis_name="s")

    @functools.partial(
        pl.kernel,
        mesh=mesh,
        out_type=jax.ShapeDtypeStruct((_PAIRS,), jnp.float32),
        scratch_types=[
            pltpu.VMEM((per_w,), jnp.int32),
            pltpu.VMEM((per_w, _D), jnp.float32),
            pltpu.VMEM((chunk, _D), jnp.float32),
            pltpu.VMEM((chunk, _D), jnp.float32),
            pltpu.VMEM((per_w,), jnp.float32),
            pltpu.SemaphoreType.DMA,
            pltpu.SemaphoreType.DMA,
        ],
        compiler_params=pltpu.CompilerParams(use_tc_tiling_on_sc=False),
    )
    def sc_dots(gid_hbm, feat_hbm, table_hbm, out_hbm,
                idx_v, feat_v, rows0_v, rows1_v, dots_v, sem0, sem1):
        wid = lax.axis_index("s") * info.num_cores + lax.axis_index("c")
        base = wid * per_w
        pltpu.sync_copy(gid_hbm.at[pl.ds(base, per_w)], idx_v)
        rows_bufs = (rows0_v, rows1_v)
        sems = (sem0, sem1)

        def fire(c):
            return pltpu.async_copy(
                table_hbm.at[idx_v.at[pl.ds(c * chunk, chunk)]],
                rows_bufs[c % 2], sems[c % 2])

        cps = {0: fire(0), 1: fire(1)}
        pltpu.sync_copy(feat_hbm.at[pl.ds(base, per_w)], feat_v)

        lane = lax.iota(jnp.int32, 16)

        def xl_gather(x, idx):
            return lax.gather(
                x, idx[:, None],
                dimension_numbers=lax.GatherDimensionNumbers(
                    offset_dims=(), collapsed_slice_dims=(0,),
                    start_index_map=(0,)),
                slice_sizes=(1,),
                mode=lax.GatherScatterMode.PROMISE_IN_BOUNDS)

        bfly = [lane ^ 8, lane ^ 4, lane ^ 2, lane ^ 1]

        for c in range(per_w // chunk):
            cps[c].wait()
            rows_v = rows_bufs[c % 2]

            def group_body(grp, _):
                dots16 = jnp.zeros((16,), jnp.float32)
                for j in range(16):
                    p = grp * 16 + j
                    acc = jnp.zeros((16,), jnp.float32)
                    for k in range(_D // 16):
                        w = rows_v[p, pl.ds(k * 16, 16)]
                        f = feat_v[c * chunk + p, pl.ds(k * 16, 16)]
                        acc = acc + w * f
                    for idx in bfly:
                        acc = acc + xl_gather(acc, idx)
                    dots16 = jnp.where(lane == j, acc, dots16)
                dots_v[pl.ds(c * chunk + grp * 16, 16)] = dots16
                return 0

            lax.fori_loop(0, chunk // 16, group_body, 0)
            if c + 2 < per_w // chunk:
                cps[c + 2] = fire(c + 2)

        pltpu.sync_copy(dots_v, out_hbm.at[pl.ds(base, per_w)])

    return sc_dots


def _tc_winner_body(gr_ref, gc_ref, mr_ref, mc_ref, out_ref):
    V = _V
    grow = gr_ref[0]
    gcol = gc_ref[0]
    mrow = mr_ref[0]
    mcol = mc_ref[0]
    v_lane = lax.broadcasted_iota(jnp.int32, (1, V), 1)
    j_sub = lax.broadcasted_iota(jnp.int32, (V, 1), 0)
    valid_row = (mrow > 0) & (grow < _NUM_GENES)
    valid_col = (mcol > 0) & (gcol < _NUM_GENES)
    krow = jnp.where(valid_row, grow, -1 - v_lane)
    kcol = jnp.where(valid_col, gcol, -1 - j_sub)
    jj = lax.broadcasted_iota(jnp.int32, (V, V), 0)
    vv = lax.broadcasted_iota(jnp.int32, (V, V), 1)
    hit = (kcol == krow) & (jj > vv)
    killed = jnp.any(hit, axis=0, keepdims=True)
    winner = valid_row & jnp.logical_not(killed)
    out_ref[0] = winner.astype(jnp.float32)


def _tc_combine_body(win_ref, dots_ref, b_ref, out_ref):
    prod = win_ref[...] * dots_ref[...]
    red = jnp.sum(prod, axis=2)
    out_ref[...] = red + b_ref[0, 0]


def kernel(features, positions, gene_ids, mask, W, b):
    del positions
    B, V, D = features.shape
    table = W.reshape(_NUM_GENES, _D)
    gid_flat = jnp.clip(gene_ids, 0, _NUM_GENES - 1).reshape(B * V)
    feat_flat = features.reshape(B * V, D)

    dots = _make_sc_dots()(gid_flat, feat_flat, table)

    gid_row = gene_ids.reshape(B, 1, V)
    gid_col = gene_ids.reshape(B, V, 1)
    msk_row = mask.reshape(B, 1, V)
    msk_col = mask.reshape(B, V, 1)

    winner = pl.pallas_call(
        _tc_winner_body,
        grid=(B,),
        in_specs=[
            pl.BlockSpec((1, 1, V), lambda i: (i, 0, 0)),
            pl.BlockSpec((1, V, 1), lambda i: (i, 0, 0)),
            pl.BlockSpec((1, 1, V), lambda i: (i, 0, 0)),
            pl.BlockSpec((1, V, 1), lambda i: (i, 0, 0)),
        ],
        out_specs=pl.BlockSpec((1, 1, V), lambda i: (i, 0, 0)),
        out_shape=jax.ShapeDtypeStruct((B, 1, V), jnp.float32),
    )(gid_row, gid_col, msk_row, msk_col)

    logits = pl.pallas_call(
        _tc_combine_body,
        out_shape=jax.ShapeDtypeStruct((B, 1), jnp.float32),
    )(winner, dots.reshape(B, 1, V), b.reshape(1, 1))
    return logits
